# baseline (speedup 1.0000x reference)
import functools
import jax
import jax.numpy as jnp
from jax import lax
from jax.experimental import pallas as pl
from jax.experimental.pallas import tpu as pltpu
from jax.experimental.pallas import tpu_sc as plsc

_PAD = 0
_NW = 32
_CH = 8
_NCH = 16


def _sc_copy_body(pos_hbm, out_hbm, buf0, buf1, si0, si1, so0, so1):
    wid = lax.axis_index("s") * 2 + lax.axis_index("c")
    base = wid * (_CH * _NCH)
    bufs = (buf0, buf1)
    sin = (si0, si1)
    sout = (so0, so1)

    def in_copy(c):
        return pltpu.make_async_copy(
            pos_hbm.at[pl.ds(base + c * _CH, _CH)], bufs[c % 2], sin[c % 2])

    def out_copy(c):
        return pltpu.make_async_copy(
            bufs[c % 2], out_hbm.at[pl.ds(base + c * _CH, _CH)], sout[c % 2])

    for c in range(_NCH):
        if c >= 2:
            out_copy(c - 2).wait()
        in_copy(c).start()
        if c >= 1:
            in_copy(c - 1).wait()
            out_copy(c - 1).start()
    in_copy(_NCH - 1).wait()
    out_copy(_NCH - 1).start()
    out_copy(_NCH - 2).wait()
    out_copy(_NCH - 1).wait()


def kernel(pos_emb, itemid_seq, training, masked_item_embedding):
    b, seq_len, h = pos_emb.shape

    labels = jnp.zeros((b, seq_len), jnp.int32)
    masked = jnp.zeros((b, seq_len), jnp.bool_)

    run = functools.partial(
        pl.kernel,
        out_type=jax.ShapeDtypeStruct((b, seq_len, h), pos_emb.dtype),
        mesh=plsc.VectorSubcoreMesh(core_axis_name="c", subcore_axis_name="s"),
        scratch_types=[
            pltpu.VMEM((_CH, seq_len, h), jnp.float32),
            pltpu.VMEM((_CH, seq_len, h), jnp.float32),
            pltpu.SemaphoreType.DMA,
            pltpu.SemaphoreType.DMA,
            pltpu.SemaphoreType.DMA,
            pltpu.SemaphoreType.DMA,
        ],
    )
    out = run(_sc_copy_body)(pos_emb)
    return out, labels, masked


# speedup vs baseline: 2.2577x; 2.2577x over previous
import functools
import jax
import jax.numpy as jnp
from jax import lax
from jax.experimental import pallas as pl
from jax.experimental.pallas import tpu as pltpu
from jax.experimental.pallas import tpu_sc as plsc

_PAD = 0
_NW = 32
_LANES = 16


def _sc_labels_body(item_hbm, train_hbm, lab_hbm, item_v, train_v, lab_v, sem):
    b, seq_len = item_hbm.shape
    rows_w = b // _NW
    groups = rows_w // _LANES
    wid = lax.axis_index("s") * 2 + lax.axis_index("c")
    base = wid * rows_w

    pltpu.make_async_copy(
        item_hbm.at[pl.ds(base, rows_w)], item_v, sem).start()
    pltpu.sync_copy(train_hbm, train_v)

    nz = (rows_w * seq_len) // _LANES
    zeros16 = jnp.zeros((_LANES,), jnp.int32)

    def zbody(i, carry):
        lab_v[pl.ds(i * _LANES, _LANES)] = zeros16
        return carry

    lax.fori_loop(0, nz, zbody, 0)

    pltpu.make_async_copy(
        item_hbm.at[pl.ds(base, rows_w)], item_v, sem).wait()

    train_ok = train_v[...] == 0
    iota16 = lax.iota(jnp.int32, _LANES)

    for g in range(groups):
        rows = g * _LANES + iota16

        def jbody(j, cnt):
            col = plsc.load_gather(
                item_v, [rows, jnp.full((_LANES,), j, jnp.int32)])
            return cnt + (col != _PAD).astype(jnp.int32)

        cnt = lax.fori_loop(0, seq_len, jbody, jnp.zeros((_LANES,), jnp.int32))
        idx = cnt - 1
        idxc = jnp.maximum(idx, 0)
        lv = plsc.load_gather(item_v, [rows, idxc])
        val = jnp.where(train_ok, lv, 0)
        plsc.store_scatter(lab_v, [rows * seq_len + idxc], val)

    pltpu.make_async_copy(
        lab_v, lab_hbm.at[pl.ds(base * seq_len, rows_w * seq_len)], sem).start()
    pltpu.make_async_copy(
        lab_v, lab_hbm.at[pl.ds(base * seq_len, rows_w * seq_len)], sem).wait()


def kernel(pos_emb, itemid_seq, training, masked_item_embedding):
    b, seq_len, h = pos_emb.shape
    rows_w = b // _NW
    train16 = jnp.full((_LANES,), training, jnp.int32)

    run = functools.partial(
        pl.kernel,
        out_type=jax.ShapeDtypeStruct((b * seq_len,), jnp.int32),
        mesh=plsc.VectorSubcoreMesh(core_axis_name="c", subcore_axis_name="s"),
        compiler_params=pltpu.CompilerParams(needs_layout_passes=False),
        scratch_types=[
            pltpu.VMEM((rows_w, seq_len), jnp.int32),
            pltpu.VMEM((_LANES,), jnp.int32),
            pltpu.VMEM((rows_w * seq_len,), jnp.int32),
            pltpu.SemaphoreType.DMA,
        ],
    )
    labels_flat = run(_sc_labels_body)(itemid_seq, train16)
    labels = labels_flat.reshape(b, seq_len)
    masked = labels != _PAD
    out = jnp.where(
        masked[..., None], masked_item_embedding.astype(pos_emb.dtype), pos_emb)
    return out, labels, masked


# speedup vs baseline: 2.3119x; 1.0240x over previous
import functools
import jax
import jax.numpy as jnp
from jax import lax
from jax.experimental import pallas as pl
from jax.experimental.pallas import tpu as pltpu
from jax.experimental.pallas import tpu_sc as plsc

_PAD = 0
_NW = 32
_LANES = 16


def _sc_labels_body(item_hbm, train_hbm, lab_hbm, item_v, train_v, lab_v, sem):
    b, seq_len = item_hbm.shape
    rows_w = b // _NW
    groups = rows_w // _LANES
    wid = lax.axis_index("s") * 2 + lax.axis_index("c")
    base = wid * rows_w

    pltpu.make_async_copy(
        item_hbm.at[pl.ds(base, rows_w)], item_v, sem).start()
    pltpu.sync_copy(train_hbm, train_v)

    nz = (rows_w * seq_len) // _LANES
    zeros16 = jnp.zeros((_LANES,), jnp.int32)
    for i in range(nz):
        lab_v[pl.ds(i * _LANES, _LANES)] = zeros16

    pltpu.make_async_copy(
        item_hbm.at[pl.ds(base, rows_w)], item_v, sem).wait()

    train_ok = train_v[...] == 0
    iota16 = lax.iota(jnp.int32, _LANES)

    for g in range(groups):
        rows = g * _LANES + iota16
        cnt = jnp.zeros((_LANES,), jnp.int32)
        for j in range(seq_len):
            col = plsc.load_gather(
                item_v, [rows, jnp.full((_LANES,), j, jnp.int32)])
            cnt = cnt + (col != _PAD).astype(jnp.int32)
        idx = cnt - 1
        idxc = jnp.maximum(idx, 0)
        lv = plsc.load_gather(item_v, [rows, idxc])
        val = jnp.where(train_ok, lv, 0)
        plsc.store_scatter(lab_v, [rows * seq_len + idxc], val)

    pltpu.make_async_copy(
        lab_v, lab_hbm.at[pl.ds(base * seq_len, rows_w * seq_len)], sem).start()
    pltpu.make_async_copy(
        lab_v, lab_hbm.at[pl.ds(base * seq_len, rows_w * seq_len)], sem).wait()


def kernel(pos_emb, itemid_seq, training, masked_item_embedding):
    b, seq_len, h = pos_emb.shape
    rows_w = b // _NW
    train16 = jnp.full((_LANES,), training, jnp.int32)

    run = functools.partial(
        pl.kernel,
        out_type=jax.ShapeDtypeStruct((b * seq_len,), jnp.int32),
        mesh=plsc.VectorSubcoreMesh(core_axis_name="c", subcore_axis_name="s"),
        compiler_params=pltpu.CompilerParams(needs_layout_passes=False),
        scratch_types=[
            pltpu.VMEM((rows_w, seq_len), jnp.int32),
            pltpu.VMEM((_LANES,), jnp.int32),
            pltpu.VMEM((rows_w * seq_len,), jnp.int32),
            pltpu.SemaphoreType.DMA,
        ],
    )
    labels_flat = run(_sc_labels_body)(itemid_seq, train16)
    labels = labels_flat.reshape(b, seq_len)
    masked = labels != _PAD
    out = jnp.where(
        masked[..., None], masked_item_embedding.astype(pos_emb.dtype), pos_emb)
    return out, labels, masked


# speedup vs baseline: 2.3534x; 1.0179x over previous
import functools
import jax
import jax.numpy as jnp
from jax import lax
from jax.experimental import pallas as pl
from jax.experimental.pallas import tpu as pltpu
from jax.experimental.pallas import tpu_sc as plsc

_PAD = 0
_NW = 32
_LANES = 16


def _sc_labels_body(item_hbm, train_hbm, lab_hbm, item_v, train_v, lab_v, sem):
    b, seq_len = item_hbm.shape
    rows_w = b // _NW
    groups = rows_w // _LANES
    wid = lax.axis_index("s") * 2 + lax.axis_index("c")
    base = wid * rows_w

    pltpu.make_async_copy(
        item_hbm.at[pl.ds(base, rows_w)], item_v, sem).start()
    pltpu.sync_copy(train_hbm, train_v)

    zeros16 = jnp.zeros((_LANES,), jnp.int32)
    starts = list(range(0, seq_len - _LANES, _LANES)) + [seq_len - _LANES]
    for r in range(rows_w):
        for s in starts:
            lab_v[r, pl.ds(s, _LANES)] = zeros16

    pltpu.make_async_copy(
        item_hbm.at[pl.ds(base, rows_w)], item_v, sem).wait()

    train_ok = train_v[...] == 0
    iota16 = lax.iota(jnp.int32, _LANES)

    for g in range(groups):
        rows = g * _LANES + iota16
        cnt = jnp.zeros((_LANES,), jnp.int32)
        for j in range(seq_len):
            col = plsc.load_gather(
                item_v, [rows, jnp.full((_LANES,), j, jnp.int32)])
            cnt = cnt + (col != _PAD).astype(jnp.int32)
        idx = cnt - 1
        idxc = jnp.maximum(idx, 0)
        lv = plsc.load_gather(item_v, [rows, idxc])
        val = jnp.where(train_ok, lv, 0)
        plsc.store_scatter(lab_v, [rows, idxc], val)

    pltpu.make_async_copy(
        lab_v, lab_hbm.at[pl.ds(base, rows_w)], sem).start()
    pltpu.make_async_copy(
        lab_v, lab_hbm.at[pl.ds(base, rows_w)], sem).wait()


def kernel(pos_emb, itemid_seq, training, masked_item_embedding):
    b, seq_len, h = pos_emb.shape
    rows_w = b // _NW
    train16 = jnp.full((_LANES,), training, jnp.int32)

    run = functools.partial(
        pl.kernel,
        out_type=jax.ShapeDtypeStruct((b, seq_len), jnp.int32),
        mesh=plsc.VectorSubcoreMesh(core_axis_name="c", subcore_axis_name="s"),
        compiler_params=pltpu.CompilerParams(needs_layout_passes=False),
        scratch_types=[
            pltpu.VMEM((rows_w, seq_len), jnp.int32),
            pltpu.VMEM((_LANES,), jnp.int32),
            pltpu.VMEM((rows_w, seq_len), jnp.int32),
            pltpu.SemaphoreType.DMA,
        ],
    )
    labels = run(_sc_labels_body)(itemid_seq, train16)
    masked = labels != _PAD
    out = jnp.where(
        masked[..., None], masked_item_embedding.astype(pos_emb.dtype), pos_emb)
    return out, labels, masked
